# baseline (speedup 1.0000x reference)
import dataclasses
import functools

import jax
import jax.numpy as jnp
from jax import lax
from jax.experimental import pallas as pl
from jax.experimental.pallas import tpu as pltpu
from jax.experimental.pallas import tpu_sc as plsc

_NC = 2
_NS = 16
_W = 64


def _xa_body(x_ref, w_ref, o_ref):
    o_ref[...] = jnp.dot(x_ref[...], w_ref[...],
                         preferred_element_type=jnp.float32,
                         precision=lax.Precision.HIGHEST)


def _tc_xa(x, W1a):
    n, fx = x.shape
    h = W1a.shape[1]
    return pl.pallas_call(
        _xa_body,
        out_shape=jax.ShapeDtypeStruct((n, h), jnp.float32),
    )(x, W1a)


def _edge_lin_body(ea_ref, w_ref, b_ref, o_ref):
    o_ref[...] = jnp.dot(ea_ref[...], w_ref[...],
                         preferred_element_type=jnp.float32,
                         precision=lax.Precision.HIGHEST) + b_ref[...]


def _tc_edge_lin(ea2, W1big, b1big):
    e8, fp = ea2.shape
    hp = W1big.shape[1]
    be = 1256
    return pl.pallas_call(
        _edge_lin_body,
        grid=(e8 // be,),
        in_specs=[
            pl.BlockSpec((be, fp), lambda i: (i, 0)),
            pl.BlockSpec((fp, hp), lambda i: (0, 0)),
            pl.BlockSpec((1, hp), lambda i: (0, 0)),
        ],
        out_specs=pl.BlockSpec((be, hp), lambda i: (i, 0)),
        out_shape=jax.ShapeDtypeStruct((e8, hp), jnp.float32),
    )(ea2, W1big, b1big.reshape(1, hp))


def _final_body(x_ref, hs0_ref, hs1_ref, cnt_ref, w2_ref, b2_ref,
                w3a_ref, w3b_ref, b3_ref, w4_ref, b4_ref, o_ref):
    hs = hs0_ref[...] + hs1_ref[...]
    cnt = lax.dot_general(cnt_ref[...], jnp.ones((32, 1), jnp.float32),
                          dimension_numbers=(((0,), (0,)), ((), ())),
                          preferred_element_type=jnp.float32,
                          precision=lax.Precision.HIGHEST)
    summed = jnp.dot(hs, w2_ref[...], preferred_element_type=jnp.float32,
                     precision=lax.Precision.HIGHEST) + cnt * b2_ref[...]
    mean = summed / jnp.maximum(cnt, 1.0)
    t = (jnp.dot(x_ref[...], w3a_ref[...], preferred_element_type=jnp.float32,
                 precision=lax.Precision.HIGHEST)
         + jnp.dot(mean, w3b_ref[...], preferred_element_type=jnp.float32,
                   precision=lax.Precision.HIGHEST)
         + b3_ref[...])
    t = jnp.maximum(t, 0.0)
    o_ref[...] = jnp.dot(t, w4_ref[...], preferred_element_type=jnp.float32,
                         precision=lax.Precision.HIGHEST) + b4_ref[...]


def _tc_final(x, hs0, hs1, cnt, W2, b2, W3a, W3b, b3, W4, b4):
    n, fx = x.shape
    h = W2.shape[0]
    bn = 2048
    grid = (n + bn - 1) // bn
    full = lambda r, c: pl.BlockSpec((r, c), lambda i: (0, 0))
    return pl.pallas_call(
        _final_body,
        grid=(grid,),
        in_specs=[
            pl.BlockSpec((bn, fx), lambda i: (i, 0)),
            pl.BlockSpec((bn, h), lambda i: (i, 0)),
            pl.BlockSpec((bn, h), lambda i: (i, 0)),
            pl.BlockSpec((32, bn), lambda i: (0, i)),
            full(h, h),
            full(1, h),
            full(fx, h),
            full(h, h),
            full(1, h),
            full(h, fx),
            full(1, fx),
        ],
        out_specs=pl.BlockSpec((bn, fx), lambda i: (i, 0)),
        out_shape=jax.ShapeDtypeStruct((n, fx), jnp.float32),
    )(x, hs0, hs1, cnt, W2, b2.reshape(1, h), W3a, W3b, b3.reshape(1, h),
      W4, b4.reshape(1, fx))


def _sc_edge(xa, ea128, rc2):
    n, h = xa.shape
    zrows = 32
    npad = ((n + _NS * zrows - 1) // (_NS * zrows)) * (_NS * zrows)
    per_worker = rc2.shape[1] * _W
    n_win = rc2.shape[1]
    per_tile_n = npad // _NS
    assert per_tile_n % zrows == 0

    mesh = plsc.VectorSubcoreMesh(core_axis_name="c", subcore_axis_name="s",
                                  num_cores=_NC, num_subcores=_NS)
    cp = pltpu.CompilerParams()
    if "needs_layout_passes" in pltpu.CompilerParams.__dataclass_fields__:
        cp = dataclasses.replace(cp, needs_layout_passes=False)

    @functools.partial(
        pl.kernel,
        compiler_params=cp,
        out_type=[
            jax.ShapeDtypeStruct((_NC, npad, h), jnp.float32),
            jax.ShapeDtypeStruct((_NC * _NS, npad), jnp.float32),
        ],
        mesh=mesh,
        scratch_types=[
            pltpu.VMEM((4, 2, _W), jnp.int32),
            pltpu.VMEM((_W, h), jnp.float32),
            pltpu.VMEM((_W, h), jnp.float32),
            pltpu.VMEM((_W // 8, 8 * h), jnp.float32),
            pltpu.VMEM((zrows, h), jnp.float32),
            pltpu.VMEM((npad,), jnp.float32),
            pltpu.VMEM_SHARED((npad, h), jnp.float32),
            pltpu.SemaphoreType.DMA,
            pltpu.SemaphoreType.DMA,
            pltpu.SemaphoreType.DMA,
            pltpu.SemaphoreType.DMA,
            pltpu.SemaphoreType.DMA,
            pltpu.SemaphoreType.DMA,
            pltpu.SemaphoreType.DMA,
            pltpu.SemaphoreType.DMA,
            pltpu.SemaphoreType.DMA,
        ],
    )
    def sc_kernel(xa_hbm, ea_hbm, rc_hbm, hs_out, cnt_out,
                  idx, gbuf0, gbuf1, ebuf, zbuf, cntbuf, hs_acc,
                  isem0, isem1, isem2, isem3, gsem0, gsem1, esem,
                  ssem0, ssem1):
        core = lax.axis_index("c")
        sub = lax.axis_index("s")
        gbufs = (gbuf0, gbuf1)
        isems = (isem0, isem1, isem2, isem3)
        gsems = (gsem0, gsem1)
        ssems = (ssem0, ssem1)

        zero16 = jnp.zeros((16,), jnp.float32)
        one16 = jnp.ones((16,), jnp.float32)

        wid = core * _NS + sub
        nbase = sub * per_tile_n
        base = wid * per_worker

        def stage_idx(w, s):
            pltpu.async_copy(rc_hbm.at[wid, w], idx.at[s], isems[s])

        def wait_idx(s):
            pltpu.make_async_copy(rc_hbm.at[wid, 0], idx.at[s],
                                  isems[s]).wait()

        def stage_gather(w, s, b):
            pltpu.async_copy(xa_hbm.at[idx.at[s, 1]], gbufs[b], gsems[b])

        def wait_gather(s, b):
            pltpu.make_async_copy(xa_hbm.at[idx.at[s, 1]], gbufs[b],
                                  gsems[b]).wait()

        rw = _W // 8
        rbase = wid * (per_worker // 8)

        def stage_ea(w):
            pltpu.async_copy(ea_hbm.at[pl.ds(rbase + w * rw, rw)], ebuf, esem)

        def wait_ea():
            pltpu.make_async_copy(ea_hbm.at[pl.ds(rbase, rw)], ebuf,
                                  esem).wait()

        def wait_scatter(s, b):
            pltpu.make_async_copy(gbufs[b], hs_acc.at[idx.at[s, 0]],
                                  ssems[b]).wait()

        stage_idx(0, 0)
        stage_idx(1, 1)

        @pl.loop(0, zrows)
        def _(i):
            for j in range(h // 16):
                zbuf[i, pl.ds(j * 16, 16)] = zero16

        @pl.loop(0, npad // 16)
        def _(i):
            cntbuf[pl.ds(i * 16, 16)] = zero16

        for kk in range(per_tile_n // zrows):
            pltpu.sync_copy(zbuf, hs_acc.at[pl.ds(nbase + kk * zrows, zrows)])

        wait_idx(0)
        stage_gather(0, 0, 0)
        stage_ea(0)
        plsc.subcore_barrier()

        def body(w, s, b, first=False, has_next=True, has_next2=True):
            if has_next:
                wait_idx((s + 1) % 4)
            wait_gather(s, b)
            wait_ea()
            if not first:
                wait_scatter((s + 3) % 4, 1 - b)
            if has_next:
                stage_gather(w + 1, (s + 1) % 4, 1 - b)
            gb = gbufs[b]

            @pl.loop(0, rw)
            def _(r):
                for c in range(8 * h // 16):
                    g = pl.ds(16 * (c % (h // 16)), 16)
                    s = pl.ds(16 * c, 16)
                    gr = r * 8 + c // (h // 16)
                    gb[gr, g] = jnp.maximum(gb[gr, g] + ebuf[r, s], 0.0)

            if has_next:
                stage_ea(w + 1)
            for k in range(_W // 16):
                iv = idx[s, 0, pl.ds(k * 16, 16)]
                plsc.addupdate_scatter(cntbuf, [iv], one16)
            pltpu.async_copy(gb, hs_acc.at[idx.at[s, 0]], ssems[b], add=True)
            if has_next2:
                stage_idx(w + 2, (s + 2) % 4)

        assert n_win >= 7 and (n_win - 2) % 4 == 3
        body(0, 0, 0, first=True)
        body(1, 1, 1)

        @pl.loop(0, (n_win - 5) // 4)
        def _(k):
            w0 = 2 + 4 * k
            body(w0 + 0, 2, 0)
            body(w0 + 1, 3, 1)
            body(w0 + 2, 0, 0)
            body(w0 + 3, 1, 1)

        body(n_win - 3, 2, 0)
        body(n_win - 2, 3, 1, has_next2=False)
        body(n_win - 1, 0, 0, has_next=False, has_next2=False)
        wait_scatter(0, 0)

        pltpu.sync_copy(cntbuf, cnt_out.at[wid])
        plsc.subcore_barrier()
        pltpu.sync_copy(hs_acc.at[pl.ds(nbase, per_tile_n)],
                        hs_out.at[core, pl.ds(nbase, per_tile_n)])

    return sc_kernel(xa, ea128, rc2)


def kernel(x, edge_index, edge_attr, u, batch, W1, b1, W2, b2, W3, b3, W4, b4):
    n, fx = x.shape
    e = edge_attr.shape[0]
    fe = edge_attr.shape[1]
    nw = _NC * _NS
    epad = -(-e // (nw * _W)) * (nw * _W)
    row = edge_index[0].astype(jnp.int32)
    col = edge_index[1].astype(jnp.int32)
    if epad != e:
        row = jnp.concatenate([row, jnp.full((epad - e,), n, jnp.int32)])
        col = jnp.concatenate([col, jnp.zeros((epad - e,), jnp.int32)])
        edge_attr = jnp.concatenate(
            [edge_attr, jnp.zeros((epad - e, fe), edge_attr.dtype)])
    rc2 = jnp.stack([row.reshape(nw, epad // (nw * _W), _W),
                     col.reshape(nw, epad // (nw * _W), _W)], axis=2)
    W1a, W1b = W1[:fx], W1[fx:]
    W3a, W3b = W3[:fx], W3[fx:]

    ea2 = edge_attr.reshape(epad // 8, 8 * fe)
    W1big = jnp.kron(jnp.eye(8, dtype=W1b.dtype), W1b)
    b1big = jnp.tile(b1, 8)

    xa = _tc_xa(x, W1a)
    ea128 = _tc_edge_lin(ea2, W1big, b1big)
    hs_parts, cnt_tiles = _sc_edge(xa, ea128, rc2)
    return _tc_final(x, hs_parts[0], hs_parts[1], cnt_tiles,
                     W2, b2, W3a, W3b, b3, W4, b4)


# speedup vs baseline: 2.1293x; 2.1293x over previous
import dataclasses
import functools

import jax
import jax.numpy as jnp
from jax import lax
from jax.experimental import pallas as pl
from jax.experimental.pallas import tpu as pltpu
from jax.experimental.pallas import tpu_sc as plsc

_NC = 2
_NS = 16
_W = 80


def _xa_body(x_ref, w_ref, o_ref):
    o_ref[...] = jnp.dot(x_ref[...], w_ref[...],
                         preferred_element_type=jnp.float32,
                         precision=lax.Precision.HIGHEST)


def _tc_xa(x, W1a):
    n, fx = x.shape
    h = W1a.shape[1]
    return pl.pallas_call(
        _xa_body,
        out_shape=jax.ShapeDtypeStruct((n, h), jnp.float32),
    )(x, W1a)


def _edge_lin_body(eat_ref, w_ref, b_ref, o_ref):
    o_ref[...] = lax.dot_general(eat_ref[...], w_ref[...],
                                 dimension_numbers=(((0,), (0,)), ((), ())),
                                 preferred_element_type=jnp.float32,
                                 precision=lax.Precision.HIGHEST) + b_ref[...]


def _tc_edge_lin(eaT, W1b, b1):
    fe, e = eaT.shape
    h = W1b.shape[1]
    be = 6400
    return pl.pallas_call(
        _edge_lin_body,
        grid=(e // be,),
        in_specs=[
            pl.BlockSpec((fe, be), lambda i: (0, i)),
            pl.BlockSpec((fe, h), lambda i: (0, 0)),
            pl.BlockSpec((1, h), lambda i: (0, 0)),
        ],
        out_specs=pl.BlockSpec((be, h), lambda i: (i, 0)),
        out_shape=jax.ShapeDtypeStruct((e, h), jnp.float32),
    )(eaT, W1b, b1.reshape(1, h))


def _final_body(x_ref, hs0_ref, hs1_ref, cnt_ref, w2_ref, b2_ref,
                w3a_ref, w3b_ref, b3_ref, w4_ref, b4_ref, o_ref):
    hs = hs0_ref[...] + hs1_ref[...]
    cnt = lax.dot_general(cnt_ref[...], jnp.ones((32, 1), jnp.float32),
                          dimension_numbers=(((0,), (0,)), ((), ())),
                          preferred_element_type=jnp.float32,
                          precision=lax.Precision.HIGHEST)
    summed = jnp.dot(hs, w2_ref[...], preferred_element_type=jnp.float32,
                     precision=lax.Precision.HIGHEST) + cnt * b2_ref[...]
    mean = summed / jnp.maximum(cnt, 1.0)
    t = (jnp.dot(x_ref[...], w3a_ref[...], preferred_element_type=jnp.float32,
                 precision=lax.Precision.HIGHEST)
         + jnp.dot(mean, w3b_ref[...], preferred_element_type=jnp.float32,
                   precision=lax.Precision.HIGHEST)
         + b3_ref[...])
    t = jnp.maximum(t, 0.0)
    o_ref[...] = jnp.dot(t, w4_ref[...], preferred_element_type=jnp.float32,
                         precision=lax.Precision.HIGHEST) + b4_ref[...]


def _tc_final(x, hs0, hs1, cnt, W2, b2, W3a, W3b, b3, W4, b4):
    n, fx = x.shape
    h = W2.shape[0]
    bn = 2048
    grid = (n + bn - 1) // bn
    full = lambda r, c: pl.BlockSpec((r, c), lambda i: (0, 0))
    return pl.pallas_call(
        _final_body,
        grid=(grid,),
        in_specs=[
            pl.BlockSpec((bn, fx), lambda i: (i, 0)),
            pl.BlockSpec((bn, h), lambda i: (i, 0)),
            pl.BlockSpec((bn, h), lambda i: (i, 0)),
            pl.BlockSpec((32, bn), lambda i: (0, i)),
            full(h, h),
            full(1, h),
            full(fx, h),
            full(h, h),
            full(1, h),
            full(h, fx),
            full(1, fx),
        ],
        out_specs=pl.BlockSpec((bn, fx), lambda i: (i, 0)),
        out_shape=jax.ShapeDtypeStruct((n, fx), jnp.float32),
    )(x, hs0, hs1, cnt, W2, b2.reshape(1, h), W3a, W3b, b3.reshape(1, h),
      W4, b4.reshape(1, fx))


def _sc_edge(xa, ea128, rc2):
    n, h = xa.shape
    zrows = 32
    npad = ((n + _NS * zrows - 1) // (_NS * zrows)) * (_NS * zrows)
    per_worker = rc2.shape[1] * _W
    n_win = rc2.shape[1]
    per_tile_n = npad // _NS
    assert per_tile_n % zrows == 0

    mesh = plsc.VectorSubcoreMesh(core_axis_name="c", subcore_axis_name="s",
                                  num_cores=_NC, num_subcores=_NS)
    cp = pltpu.CompilerParams()
    if "needs_layout_passes" in pltpu.CompilerParams.__dataclass_fields__:
        cp = dataclasses.replace(cp, needs_layout_passes=False)

    @functools.partial(
        pl.kernel,
        compiler_params=cp,
        out_type=[
            jax.ShapeDtypeStruct((_NC, npad, h), jnp.float32),
            jax.ShapeDtypeStruct((_NC * _NS, npad), jnp.float32),
        ],
        mesh=mesh,
        scratch_types=[
            pltpu.VMEM((4, 2, _W), jnp.int32),
            pltpu.VMEM((_W, h), jnp.float32),
            pltpu.VMEM((_W, h), jnp.float32),
            pltpu.VMEM((_W, h), jnp.float32),
            pltpu.VMEM((zrows, h), jnp.float32),
            pltpu.VMEM((npad,), jnp.float32),
            pltpu.VMEM_SHARED((npad, h), jnp.float32),
            pltpu.SemaphoreType.DMA,
            pltpu.SemaphoreType.DMA,
            pltpu.SemaphoreType.DMA,
            pltpu.SemaphoreType.DMA,
            pltpu.SemaphoreType.DMA,
            pltpu.SemaphoreType.DMA,
            pltpu.SemaphoreType.DMA,
            pltpu.SemaphoreType.DMA,
            pltpu.SemaphoreType.DMA,
        ],
    )
    def sc_kernel(xa_hbm, ea_hbm, rc_hbm, hs_out, cnt_out,
                  idx, gbuf0, gbuf1, ebuf, zbuf, cntbuf, hs_acc,
                  isem0, isem1, isem2, isem3, gsem0, gsem1, esem,
                  ssem0, ssem1):
        core = lax.axis_index("c")
        sub = lax.axis_index("s")
        gbufs = (gbuf0, gbuf1)
        isems = (isem0, isem1, isem2, isem3)
        gsems = (gsem0, gsem1)
        ssems = (ssem0, ssem1)

        zero16 = jnp.zeros((16,), jnp.float32)
        one16 = jnp.ones((16,), jnp.float32)

        wid = core * _NS + sub
        nbase = sub * per_tile_n
        base = wid * per_worker

        def stage_idx(w, s):
            pltpu.async_copy(rc_hbm.at[wid, w], idx.at[s], isems[s])

        def wait_idx(s):
            pltpu.make_async_copy(rc_hbm.at[wid, 0], idx.at[s],
                                  isems[s]).wait()

        def stage_gather(w, s, b):
            pltpu.async_copy(xa_hbm.at[idx.at[s, 1]], gbufs[b], gsems[b])

        def wait_gather(s, b):
            pltpu.make_async_copy(xa_hbm.at[idx.at[s, 1]], gbufs[b],
                                  gsems[b]).wait()

        def stage_ea(w):
            pltpu.async_copy(ea_hbm.at[pl.ds(base + w * _W, _W)], ebuf, esem)

        def wait_ea():
            pltpu.make_async_copy(ea_hbm.at[pl.ds(base, _W)], ebuf,
                                  esem).wait()

        def wait_scatter(s, b):
            pltpu.make_async_copy(gbufs[b], hs_acc.at[idx.at[s, 0]],
                                  ssems[b]).wait()

        stage_idx(0, 0)
        stage_idx(1, 1)

        @pl.loop(0, zrows)
        def _(i):
            for j in range(h // 16):
                zbuf[i, pl.ds(j * 16, 16)] = zero16

        @pl.loop(0, npad // 16)
        def _(i):
            cntbuf[pl.ds(i * 16, 16)] = zero16

        for kk in range(per_tile_n // zrows):
            pltpu.sync_copy(zbuf, hs_acc.at[pl.ds(nbase + kk * zrows, zrows)])

        wait_idx(0)
        stage_gather(0, 0, 0)
        stage_ea(0)
        plsc.subcore_barrier()

        def body(w, s, b, first=False, has_next=True, has_next2=True):
            if has_next:
                wait_idx((s + 1) % 4)
            wait_gather(s, b)
            wait_ea()
            if not first:
                wait_scatter((s + 3) % 4, 1 - b)
            if has_next:
                stage_gather(w + 1, (s + 1) % 4, 1 - b)
            gb = gbufs[b]

            @pl.loop(0, _W, step=2)
            def _(i):
                for ii in range(2):
                    for j in range(h // 16):
                        c = pl.ds(j * 16, 16)
                        gb[i + ii, c] = jnp.maximum(gb[i + ii, c]
                                                    + ebuf[i + ii, c], 0.0)

            if has_next:
                stage_ea(w + 1)
            for k in range(_W // 16):
                iv = idx[s, 0, pl.ds(k * 16, 16)]
                plsc.addupdate_scatter(cntbuf, [iv], one16)
            pltpu.async_copy(gb, hs_acc.at[idx.at[s, 0]], ssems[b], add=True)
            if has_next2:
                stage_idx(w + 2, (s + 2) % 4)

        assert n_win >= 7 and (n_win - 2) % 4 == 3
        body(0, 0, 0, first=True)
        body(1, 1, 1)

        @pl.loop(0, (n_win - 5) // 4)
        def _(k):
            w0 = 2 + 4 * k
            body(w0 + 0, 2, 0)
            body(w0 + 1, 3, 1)
            body(w0 + 2, 0, 0)
            body(w0 + 3, 1, 1)

        body(n_win - 3, 2, 0)
        body(n_win - 2, 3, 1, has_next2=False)
        body(n_win - 1, 0, 0, has_next=False, has_next2=False)
        wait_scatter(0, 0)

        pltpu.sync_copy(cntbuf, cnt_out.at[wid])
        plsc.subcore_barrier()
        pltpu.sync_copy(hs_acc.at[pl.ds(nbase, per_tile_n)],
                        hs_out.at[core, pl.ds(nbase, per_tile_n)])

    return sc_kernel(xa, ea128, rc2)


def kernel(x, edge_index, edge_attr, u, batch, W1, b1, W2, b2, W3, b3, W4, b4):
    n, fx = x.shape
    e = edge_attr.shape[0]
    nw = _NC * _NS
    rc2 = jnp.stack(
        [edge_index[0].astype(jnp.int32).reshape(nw, e // (nw * _W), _W),
         edge_index[1].astype(jnp.int32).reshape(nw, e // (nw * _W), _W)],
        axis=2)
    W1a, W1b = W1[:fx], W1[fx:]
    W3a, W3b = W3[:fx], W3[fx:]

    xa = _tc_xa(x, W1a)
    ea128 = _tc_edge_lin(edge_attr.T, W1b, b1)
    hs_parts, cnt_tiles = _sc_edge(xa, ea128, rc2)
    return _tc_final(x, hs_parts[0], hs_parts[1], cnt_tiles,
                     W2, b2, W3a, W3b, b3, W4, b4)


# speedup vs baseline: 2.4586x; 1.1546x over previous
import dataclasses
import functools

import jax
import jax.numpy as jnp
from jax import lax
from jax.experimental import pallas as pl
from jax.experimental.pallas import tpu as pltpu
from jax.experimental.pallas import tpu_sc as plsc

_NC = 2
_NS = 16
_W = 80


def _xa_body(x_ref, w_ref, o_ref):
    o_ref[...] = jnp.dot(x_ref[...], w_ref[...],
                         preferred_element_type=jnp.float32,
                         precision=lax.Precision.HIGHEST)


def _tc_xa(x, W1a):
    n, fx = x.shape
    h = W1a.shape[1]
    return pl.pallas_call(
        _xa_body,
        out_shape=jax.ShapeDtypeStruct((n, h), jnp.float32),
    )(x, W1a)


def _edge_lin_body(eat_ref, w_ref, b_ref, o_ref):
    o_ref[...] = lax.dot_general(eat_ref[...], w_ref[...],
                                 dimension_numbers=(((0,), (0,)), ((), ())),
                                 preferred_element_type=jnp.float32,
                                 precision=lax.Precision.DEFAULT) + b_ref[...]


def _tc_edge_lin(eaT, W1b, b1):
    fe, e = eaT.shape
    h = W1b.shape[1]
    be = 6400
    return pl.pallas_call(
        _edge_lin_body,
        grid=(e // be,),
        in_specs=[
            pl.BlockSpec((fe, be), lambda i: (0, i)),
            pl.BlockSpec((fe, h), lambda i: (0, 0)),
            pl.BlockSpec((1, h), lambda i: (0, 0)),
        ],
        out_specs=pl.BlockSpec((be, h), lambda i: (i, 0)),
        out_shape=jax.ShapeDtypeStruct((e, h), jnp.float32),
    )(eaT, W1b, b1.reshape(1, h))


def _final_body(x_ref, hs0_ref, hs1_ref, cnt_ref, w2_ref, b2_ref,
                w3a_ref, w3b_ref, b3_ref, w4_ref, b4_ref, o_ref):
    hs = hs0_ref[...] + hs1_ref[...]
    cnt = lax.dot_general(cnt_ref[...], jnp.ones((32, 1), jnp.float32),
                          dimension_numbers=(((0,), (0,)), ((), ())),
                          preferred_element_type=jnp.float32,
                          precision=lax.Precision.HIGHEST)
    summed = jnp.dot(hs, w2_ref[...], preferred_element_type=jnp.float32,
                     precision=lax.Precision.HIGHEST) + cnt * b2_ref[...]
    mean = summed / jnp.maximum(cnt, 1.0)
    t = (jnp.dot(x_ref[...], w3a_ref[...], preferred_element_type=jnp.float32,
                 precision=lax.Precision.HIGHEST)
         + jnp.dot(mean, w3b_ref[...], preferred_element_type=jnp.float32,
                   precision=lax.Precision.HIGHEST)
         + b3_ref[...])
    t = jnp.maximum(t, 0.0)
    o_ref[...] = jnp.dot(t, w4_ref[...], preferred_element_type=jnp.float32,
                         precision=lax.Precision.HIGHEST) + b4_ref[...]


def _tc_final(x, hs0, hs1, cnt, W2, b2, W3a, W3b, b3, W4, b4):
    n, fx = x.shape
    h = W2.shape[0]
    bn = 2048
    grid = (n + bn - 1) // bn
    full = lambda r, c: pl.BlockSpec((r, c), lambda i: (0, 0))
    return pl.pallas_call(
        _final_body,
        grid=(grid,),
        in_specs=[
            pl.BlockSpec((bn, fx), lambda i: (i, 0)),
            pl.BlockSpec((bn, h), lambda i: (i, 0)),
            pl.BlockSpec((bn, h), lambda i: (i, 0)),
            pl.BlockSpec((32, bn), lambda i: (0, i)),
            full(h, h),
            full(1, h),
            full(fx, h),
            full(h, h),
            full(1, h),
            full(h, fx),
            full(1, fx),
        ],
        out_specs=pl.BlockSpec((bn, fx), lambda i: (i, 0)),
        out_shape=jax.ShapeDtypeStruct((n, fx), jnp.float32),
    )(x, hs0, hs1, cnt, W2, b2.reshape(1, h), W3a, W3b, b3.reshape(1, h),
      W4, b4.reshape(1, fx))


def _sc_edge(xa, ea128, rc2):
    n, h = xa.shape
    zrows = 32
    npad = ((n + _NS * zrows - 1) // (_NS * zrows)) * (_NS * zrows)
    per_worker = rc2.shape[1] * _W
    n_win = rc2.shape[1]
    per_tile_n = npad // _NS
    assert per_tile_n % zrows == 0

    mesh = plsc.VectorSubcoreMesh(core_axis_name="c", subcore_axis_name="s",
                                  num_cores=_NC, num_subcores=_NS)
    cp = pltpu.CompilerParams()
    if "needs_layout_passes" in pltpu.CompilerParams.__dataclass_fields__:
        cp = dataclasses.replace(cp, needs_layout_passes=False)

    @functools.partial(
        pl.kernel,
        compiler_params=cp,
        out_type=[
            jax.ShapeDtypeStruct((_NC, npad, h), jnp.float32),
            jax.ShapeDtypeStruct((_NC * _NS, npad), jnp.float32),
        ],
        mesh=mesh,
        scratch_types=[
            pltpu.VMEM((4, 2, _W), jnp.int32),
            pltpu.VMEM((_W, h), jnp.float32),
            pltpu.VMEM((_W, h), jnp.float32),
            pltpu.VMEM((_W, h), jnp.float32),
            pltpu.VMEM((zrows, h), jnp.float32),
            pltpu.VMEM((npad,), jnp.float32),
            pltpu.VMEM_SHARED((npad, h), jnp.float32),
            pltpu.SemaphoreType.DMA,
            pltpu.SemaphoreType.DMA,
            pltpu.SemaphoreType.DMA,
            pltpu.SemaphoreType.DMA,
            pltpu.SemaphoreType.DMA,
            pltpu.SemaphoreType.DMA,
            pltpu.SemaphoreType.DMA,
            pltpu.SemaphoreType.DMA,
            pltpu.SemaphoreType.DMA,
        ],
    )
    def sc_kernel(xa_hbm, ea_hbm, rc_hbm, hs_out, cnt_out,
                  idx, gbuf0, gbuf1, ebuf, zbuf, cntbuf, hs_acc,
                  isem0, isem1, isem2, isem3, gsem0, gsem1, esem,
                  ssem0, ssem1):
        core = lax.axis_index("c")
        sub = lax.axis_index("s")
        gbufs = (gbuf0, gbuf1)
        isems = (isem0, isem1, isem2, isem3)
        gsems = (gsem0, gsem1)
        ssems = (ssem0, ssem1)

        zero16 = jnp.zeros((16,), jnp.float32)
        one16 = jnp.ones((16,), jnp.float32)

        wid = core * _NS + sub
        nbase = sub * per_tile_n
        base = wid * per_worker

        def stage_idx(w, s):
            pltpu.async_copy(rc_hbm.at[wid, w], idx.at[s], isems[s])

        def wait_idx(s):
            pltpu.make_async_copy(rc_hbm.at[wid, 0], idx.at[s],
                                  isems[s]).wait()

        def stage_gather(w, s, b):
            pltpu.async_copy(xa_hbm.at[idx.at[s, 1]], gbufs[b], gsems[b])

        def wait_gather(s, b):
            pltpu.make_async_copy(xa_hbm.at[idx.at[s, 1]], gbufs[b],
                                  gsems[b]).wait()

        def stage_ea(w):
            pltpu.async_copy(ea_hbm.at[pl.ds(base + w * _W, _W)], ebuf, esem)

        def wait_ea():
            pltpu.make_async_copy(ea_hbm.at[pl.ds(base, _W)], ebuf,
                                  esem).wait()

        def wait_scatter(s, b):
            pltpu.make_async_copy(gbufs[b], hs_acc.at[idx.at[s, 0]],
                                  ssems[b]).wait()

        stage_idx(0, 0)
        stage_idx(1, 1)

        @pl.loop(0, zrows)
        def _(i):
            for j in range(h // 16):
                zbuf[i, pl.ds(j * 16, 16)] = zero16

        @pl.loop(0, npad // 16)
        def _(i):
            cntbuf[pl.ds(i * 16, 16)] = zero16

        for kk in range(per_tile_n // zrows):
            pltpu.sync_copy(zbuf, hs_acc.at[pl.ds(nbase + kk * zrows, zrows)])

        wait_idx(0)
        stage_gather(0, 0, 0)
        stage_ea(0)
        plsc.subcore_barrier()

        def body(w, s, b, first=False, has_next=True, has_next2=True):
            if has_next:
                wait_idx((s + 1) % 4)
            wait_gather(s, b)
            wait_ea()
            if not first:
                wait_scatter((s + 3) % 4, 1 - b)
            if has_next:
                stage_gather(w + 1, (s + 1) % 4, 1 - b)
            gb = gbufs[b]

            @pl.loop(0, _W, step=2)
            def _(i):
                for ii in range(2):
                    for j in range(h // 16):
                        c = pl.ds(j * 16, 16)
                        gb[i + ii, c] = jnp.maximum(gb[i + ii, c]
                                                    + ebuf[i + ii, c], 0.0)

            if has_next:
                stage_ea(w + 1)
            for k in range(_W // 16):
                iv = idx[s, 0, pl.ds(k * 16, 16)]
                plsc.addupdate_scatter(cntbuf, [iv], one16)
            pltpu.async_copy(gb, hs_acc.at[idx.at[s, 0]], ssems[b], add=True)
            if has_next2:
                stage_idx(w + 2, (s + 2) % 4)

        assert n_win >= 7 and (n_win - 2) % 4 == 3
        body(0, 0, 0, first=True)
        body(1, 1, 1)

        @pl.loop(0, (n_win - 5) // 4)
        def _(k):
            w0 = 2 + 4 * k
            body(w0 + 0, 2, 0)
            body(w0 + 1, 3, 1)
            body(w0 + 2, 0, 0)
            body(w0 + 3, 1, 1)

        body(n_win - 3, 2, 0)
        body(n_win - 2, 3, 1, has_next2=False)
        body(n_win - 1, 0, 0, has_next=False, has_next2=False)
        wait_scatter(0, 0)

        pltpu.sync_copy(cntbuf, cnt_out.at[wid])
        plsc.subcore_barrier()
        pltpu.sync_copy(hs_acc.at[pl.ds(nbase, per_tile_n)],
                        hs_out.at[core, pl.ds(nbase, per_tile_n)])

    return sc_kernel(xa, ea128, rc2)


def kernel(x, edge_index, edge_attr, u, batch, W1, b1, W2, b2, W3, b3, W4, b4):
    n, fx = x.shape
    e = edge_attr.shape[0]
    nw = _NC * _NS
    rc2 = jnp.stack(
        [edge_index[0].astype(jnp.int32).reshape(nw, e // (nw * _W), _W),
         edge_index[1].astype(jnp.int32).reshape(nw, e // (nw * _W), _W)],
        axis=2)
    W1a, W1b = W1[:fx], W1[fx:]
    W3a, W3b = W3[:fx], W3[fx:]

    xa = _tc_xa(x, W1a)
    ea128 = _tc_edge_lin(edge_attr.T, W1b, b1)
    hs_parts, cnt_tiles = _sc_edge(xa, ea128, rc2)
    return _tc_final(x, hs_parts[0], hs_parts[1], cnt_tiles,
                     W2, b2, W3a, W3b, b3, W4, b4)


# speedup vs baseline: 2.6532x; 1.0792x over previous
import dataclasses
import functools

import jax
import jax.numpy as jnp
from jax import lax
from jax.experimental import pallas as pl
from jax.experimental.pallas import tpu as pltpu
from jax.experimental.pallas import tpu_sc as plsc

_NC = 2
_NS = 16
_W = 80


def _xa_body(x_ref, w_ref, o_ref):
    o_ref[...] = jnp.dot(x_ref[...], w_ref[...],
                         preferred_element_type=jnp.float32,
                         precision=lax.Precision.DEFAULT)


def _tc_xa(x, W1a):
    n, fx = x.shape
    h = W1a.shape[1]
    return pl.pallas_call(
        _xa_body,
        out_shape=jax.ShapeDtypeStruct((n, h), jnp.float32),
    )(x, W1a)


def _edge_lin_body(eat_ref, w_ref, b_ref, o_ref):
    o_ref[...] = lax.dot_general(eat_ref[...], w_ref[...],
                                 dimension_numbers=(((0,), (0,)), ((), ())),
                                 preferred_element_type=jnp.float32,
                                 precision=lax.Precision.DEFAULT) + b_ref[...]


def _tc_edge_lin(eaT, W1b, b1):
    fe, e = eaT.shape
    h = W1b.shape[1]
    be = 6400
    return pl.pallas_call(
        _edge_lin_body,
        grid=(e // be,),
        in_specs=[
            pl.BlockSpec((fe, be), lambda i: (0, i)),
            pl.BlockSpec((fe, h), lambda i: (0, 0)),
            pl.BlockSpec((1, h), lambda i: (0, 0)),
        ],
        out_specs=pl.BlockSpec((be, h), lambda i: (i, 0)),
        out_shape=jax.ShapeDtypeStruct((e, h), jnp.float32),
    )(eaT, W1b, b1.reshape(1, h))


def _final_body(x_ref, hs0_ref, hs1_ref, cnt_ref, w2_ref, b2_ref,
                w3a_ref, w3b_ref, b3_ref, w4_ref, b4_ref, o_ref):
    hs = hs0_ref[...] + hs1_ref[...]
    cnt = lax.dot_general(cnt_ref[...], jnp.ones((32, 1), jnp.float32),
                          dimension_numbers=(((0,), (0,)), ((), ())),
                          preferred_element_type=jnp.float32,
                          precision=lax.Precision.DEFAULT)
    summed = jnp.dot(hs, w2_ref[...], preferred_element_type=jnp.float32,
                     precision=lax.Precision.DEFAULT) + cnt * b2_ref[...]
    mean = summed / jnp.maximum(cnt, 1.0)
    t = (jnp.dot(x_ref[...], w3a_ref[...], preferred_element_type=jnp.float32,
                 precision=lax.Precision.DEFAULT)
         + jnp.dot(mean, w3b_ref[...], preferred_element_type=jnp.float32,
                   precision=lax.Precision.DEFAULT)
         + b3_ref[...])
    t = jnp.maximum(t, 0.0)
    o_ref[...] = jnp.dot(t, w4_ref[...], preferred_element_type=jnp.float32,
                         precision=lax.Precision.DEFAULT) + b4_ref[...]


def _tc_final(x, hs0, hs1, cnt, W2, b2, W3a, W3b, b3, W4, b4):
    n, fx = x.shape
    h = W2.shape[0]
    bn = 2048
    grid = (n + bn - 1) // bn
    full = lambda r, c: pl.BlockSpec((r, c), lambda i: (0, 0))
    return pl.pallas_call(
        _final_body,
        grid=(grid,),
        in_specs=[
            pl.BlockSpec((bn, fx), lambda i: (i, 0)),
            pl.BlockSpec((bn, h), lambda i: (i, 0)),
            pl.BlockSpec((bn, h), lambda i: (i, 0)),
            pl.BlockSpec((32, bn), lambda i: (0, i)),
            full(h, h),
            full(1, h),
            full(fx, h),
            full(h, h),
            full(1, h),
            full(h, fx),
            full(1, fx),
        ],
        out_specs=pl.BlockSpec((bn, fx), lambda i: (i, 0)),
        out_shape=jax.ShapeDtypeStruct((n, fx), jnp.float32),
    )(x, hs0, hs1, cnt, W2, b2.reshape(1, h), W3a, W3b, b3.reshape(1, h),
      W4, b4.reshape(1, fx))


def _sc_edge(xa, ea128, rc2):
    n, h = xa.shape
    zrows = 32
    npad = ((n + _NS * zrows - 1) // (_NS * zrows)) * (_NS * zrows)
    per_worker = rc2.shape[1] * _W
    n_win = rc2.shape[1]
    per_tile_n = npad // _NS
    assert per_tile_n % zrows == 0

    mesh = plsc.VectorSubcoreMesh(core_axis_name="c", subcore_axis_name="s",
                                  num_cores=_NC, num_subcores=_NS)
    cp = pltpu.CompilerParams()
    if "needs_layout_passes" in pltpu.CompilerParams.__dataclass_fields__:
        cp = dataclasses.replace(cp, needs_layout_passes=False)

    @functools.partial(
        pl.kernel,
        compiler_params=cp,
        out_type=[
            jax.ShapeDtypeStruct((_NC, npad, h), jnp.float32),
            jax.ShapeDtypeStruct((_NC * _NS, npad), jnp.float32),
        ],
        mesh=mesh,
        scratch_types=[
            pltpu.VMEM((4, 2, _W), jnp.int32),
            pltpu.VMEM((_W, h), jnp.float32),
            pltpu.VMEM((_W, h), jnp.float32),
            pltpu.VMEM((_W, h), jnp.float32),
            pltpu.VMEM((zrows, h), jnp.float32),
            pltpu.VMEM((npad,), jnp.float32),
            pltpu.VMEM_SHARED((npad, h), jnp.float32),
            pltpu.SemaphoreType.DMA,
            pltpu.SemaphoreType.DMA,
            pltpu.SemaphoreType.DMA,
            pltpu.SemaphoreType.DMA,
            pltpu.SemaphoreType.DMA,
            pltpu.SemaphoreType.DMA,
            pltpu.SemaphoreType.DMA,
            pltpu.SemaphoreType.DMA,
            pltpu.SemaphoreType.DMA,
        ],
    )
    def sc_kernel(xa_hbm, ea_hbm, rc_hbm, hs_out, cnt_out,
                  idx, gbuf0, gbuf1, ebuf, zbuf, cntbuf, hs_acc,
                  isem0, isem1, isem2, isem3, gsem0, gsem1, esem,
                  ssem0, ssem1):
        core = lax.axis_index("c")
        sub = lax.axis_index("s")
        gbufs = (gbuf0, gbuf1)
        isems = (isem0, isem1, isem2, isem3)
        gsems = (gsem0, gsem1)
        ssems = (ssem0, ssem1)

        zero16 = jnp.zeros((16,), jnp.float32)
        one16 = jnp.ones((16,), jnp.float32)

        wid = core * _NS + sub
        nbase = sub * per_tile_n
        base = wid * per_worker

        def stage_idx(w, s):
            pltpu.async_copy(rc_hbm.at[wid, w], idx.at[s], isems[s])

        def wait_idx(s):
            pltpu.make_async_copy(rc_hbm.at[wid, 0], idx.at[s],
                                  isems[s]).wait()

        def stage_gather(w, s, b):
            pltpu.async_copy(xa_hbm.at[idx.at[s, 1]], gbufs[b], gsems[b])

        def wait_gather(s, b):
            pltpu.make_async_copy(xa_hbm.at[idx.at[s, 1]], gbufs[b],
                                  gsems[b]).wait()

        def stage_ea(w):
            pltpu.async_copy(ea_hbm.at[pl.ds(base + w * _W, _W)], ebuf, esem)

        def wait_ea():
            pltpu.make_async_copy(ea_hbm.at[pl.ds(base, _W)], ebuf,
                                  esem).wait()

        def wait_scatter(s, b):
            pltpu.make_async_copy(gbufs[b], hs_acc.at[idx.at[s, 0]],
                                  ssems[b]).wait()

        stage_idx(0, 0)
        stage_idx(1, 1)

        @pl.loop(0, zrows)
        def _(i):
            for j in range(h // 16):
                zbuf[i, pl.ds(j * 16, 16)] = zero16

        @pl.loop(0, npad // 16)
        def _(i):
            cntbuf[pl.ds(i * 16, 16)] = zero16

        for kk in range(per_tile_n // zrows):
            pltpu.sync_copy(zbuf, hs_acc.at[pl.ds(nbase + kk * zrows, zrows)])

        wait_idx(0)
        stage_gather(0, 0, 0)
        stage_ea(0)
        plsc.subcore_barrier()

        def body(w, s, b, first=False, has_next=True, has_next2=True):
            if has_next:
                wait_idx((s + 1) % 4)
            wait_gather(s, b)
            wait_ea()
            if not first:
                wait_scatter((s + 3) % 4, 1 - b)
            if has_next:
                stage_gather(w + 1, (s + 1) % 4, 1 - b)
            gb = gbufs[b]

            @pl.loop(0, _W, step=2)
            def _(i):
                for ii in range(2):
                    for j in range(h // 16):
                        c = pl.ds(j * 16, 16)
                        gb[i + ii, c] = jnp.maximum(gb[i + ii, c]
                                                    + ebuf[i + ii, c], 0.0)

            if has_next:
                stage_ea(w + 1)
            for k in range(_W // 16):
                iv = idx[s, 0, pl.ds(k * 16, 16)]
                plsc.addupdate_scatter(cntbuf, [iv], one16)
            pltpu.async_copy(gb, hs_acc.at[idx.at[s, 0]], ssems[b], add=True)
            if has_next2:
                stage_idx(w + 2, (s + 2) % 4)

        assert n_win >= 7 and (n_win - 2) % 4 == 3
        body(0, 0, 0, first=True)
        body(1, 1, 1)

        @pl.loop(0, (n_win - 5) // 4)
        def _(k):
            w0 = 2 + 4 * k
            body(w0 + 0, 2, 0)
            body(w0 + 1, 3, 1)
            body(w0 + 2, 0, 0)
            body(w0 + 3, 1, 1)

        body(n_win - 3, 2, 0)
        body(n_win - 2, 3, 1, has_next2=False)
        body(n_win - 1, 0, 0, has_next=False, has_next2=False)
        wait_scatter(0, 0)

        pltpu.sync_copy(cntbuf, cnt_out.at[wid])
        plsc.subcore_barrier()
        pltpu.sync_copy(hs_acc.at[pl.ds(nbase, per_tile_n)],
                        hs_out.at[core, pl.ds(nbase, per_tile_n)])

    return sc_kernel(xa, ea128, rc2)


def kernel(x, edge_index, edge_attr, u, batch, W1, b1, W2, b2, W3, b3, W4, b4):
    n, fx = x.shape
    e = edge_attr.shape[0]
    nw = _NC * _NS
    rc2 = jnp.stack(
        [edge_index[0].astype(jnp.int32).reshape(nw, e // (nw * _W), _W),
         edge_index[1].astype(jnp.int32).reshape(nw, e // (nw * _W), _W)],
        axis=2)
    W1a, W1b = W1[:fx], W1[fx:]
    W3a, W3b = W3[:fx], W3[fx:]

    xa = _tc_xa(x, W1a)
    ea128 = _tc_edge_lin(edge_attr.T, W1b, b1)
    hs_parts, cnt_tiles = _sc_edge(xa, ea128, rc2)
    return _tc_final(x, hs_parts[0], hs_parts[1], cnt_tiles,
                     W2, b2, W3a, W3b, b3, W4, b4)
